# Initial kernel scaffold; baseline (speedup 1.0000x reference)
import jax
import jax.numpy as jnp
from jax.experimental import pallas as pl


def kernel(nfeats, edge_index, efeats, emb, W_self_0, W_agg_0, b_0, W_self_1, W_agg_1, b_1, W_self_2, W_agg_2, b_2, Wr0, br0, Wr1, br1, Wr2, br2):
    raise NotImplementedError("write your pallas kernel here")


# speedup vs baseline: 8.0761x; 8.0761x over previous
import functools

import jax
import jax.numpy as jnp
from jax import lax
from jax.experimental import pallas as pl
from jax.experimental.pallas import tpu as pltpu
from jax.experimental.pallas import tpu_sc as plsc

N = 10000
E = 320000
H = 128

NW = 32
E_PER_W = E // NW
CHUNK = 125
N_CHUNKS = E_PER_W // CHUNK
N_PAD = 10240
WCHUNK = 128
ROWS_PER_TILE = N_PAD // 16
ROW_STEPS = ROWS_PER_TILE // WCHUNK

BLK = 1000
GRID = N // BLK


def _leaky_in(x):
    return jnp.where(x >= 0, x, 0.2 * x)


def _sc_body(src_hbm, dst_hbm, v_hbm, zeros_hbm, out_hbm,
             src_v, dst_v, rows, sem, acc):
    c = lax.axis_index("c")
    s = lax.axis_index("s")
    w = c * 16 + s

    pltpu.sync_copy(zeros_hbm, rows)
    for r in range(ROW_STEPS):
        pltpu.sync_copy(rows, acc.at[pl.ds(s * ROWS_PER_TILE + r * WCHUNK, WCHUNK)])
    pltpu.sync_copy(src_hbm.at[w], src_v)
    pltpu.sync_copy(dst_hbm.at[w], dst_v)
    plsc.subcore_barrier()

    def body(j, carry):
        pltpu.async_copy(v_hbm.at[src_v.at[j]], rows.at[pl.ds(0, CHUNK)], sem).wait()
        pltpu.sync_copy(rows.at[pl.ds(0, CHUNK)], acc.at[dst_v.at[j]], add=True)
        return carry

    lax.fori_loop(0, N_CHUNKS, body, 0)
    plsc.subcore_barrier()

    for r in range(ROW_STEPS):
        sl = pl.ds(s * ROWS_PER_TILE + r * WCHUNK, WCHUNK)
        pltpu.sync_copy(acc.at[sl], rows)
        pltpu.sync_copy(rows, out_hbm.at[c, sl])


@functools.cache
def _sc_segsum_fn():
    mesh = plsc.VectorSubcoreMesh(core_axis_name="c", subcore_axis_name="s")
    return pl.kernel(
        _sc_body,
        out_type=jax.ShapeDtypeStruct((2, N_PAD, H), jnp.float32),
        mesh=mesh,
        scratch_types=[
            pltpu.VMEM((N_CHUNKS, CHUNK), jnp.int32),
            pltpu.VMEM((N_CHUNKS, CHUNK), jnp.int32),
            pltpu.VMEM((WCHUNK, H), jnp.float32),
            pltpu.SemaphoreType.DMA,
            pltpu.VMEM_SHARED((N_PAD, H), jnp.float32),
        ],
    )


def _sc_segsum(src, dst, v, zeros):
    return _sc_segsum_fn()(src, dst, v, zeros)


def _tc0_body(nf_ref, emb_ref, ws_ref, wa_ref, b_ref, h_out, u_out, v_out):
    ids = lax.broadcasted_iota(jnp.int32, (BLK, H), 1)
    oh = (nf_ref[...] == ids).astype(jnp.float32)
    h0 = jnp.dot(oh, emb_ref[...], preferred_element_type=jnp.float32)
    h_out[...] = h0
    u_out[...] = jnp.dot(h0, ws_ref[...], preferred_element_type=jnp.float32) + b_ref[...]
    v_out[...] = jnp.dot(h0, wa_ref[...], preferred_element_type=jnp.float32)


def _tc_step_body(h_ref, u_ref, p_ref, q_ref, ws_ref, wa_ref, b_ref,
                  h_out, u_out, v_out):
    t = u_ref[...] + p_ref[0] + q_ref[0]
    hn = _leaky_in(t) + h_ref[...]
    h_out[...] = hn
    u_out[...] = jnp.dot(hn, ws_ref[...], preferred_element_type=jnp.float32) + b_ref[...]
    v_out[...] = jnp.dot(hn, wa_ref[...], preferred_element_type=jnp.float32)


def _tc_final_body(h_ref, u_ref, p_ref, q_ref,
                   wr0_ref, br0_ref, wr1_ref, br1_ref, wr2_ref, br2_ref,
                   out_ref, acc_ref):
    i = pl.program_id(0)
    t = u_ref[...] + p_ref[0] + q_ref[0]
    hn = _leaky_in(t) + h_ref[...]
    part = jnp.sum(hn, axis=0, keepdims=True)

    @pl.when(i == 0)
    def _():
        acc_ref[...] = part

    @pl.when(i > 0)
    def _():
        acc_ref[...] = acc_ref[...] + part

    @pl.when(i == pl.num_programs(0) - 1)
    def _():
        g = acc_ref[...]
        x = _leaky_in(jnp.dot(g, wr0_ref[...], preferred_element_type=jnp.float32) + br0_ref[...])
        x = _leaky_in(jnp.dot(x, wr1_ref[...], preferred_element_type=jnp.float32) + br1_ref[...])
        out_ref[...] = jnp.dot(x, wr2_ref[...], preferred_element_type=jnp.float32) + br2_ref[...]


def _nblk(i):
    return (i, 0)


def _const(i):
    return (0, 0)


_spec_nh = pl.BlockSpec((BLK, H), _nblk)
_spec_part0 = pl.BlockSpec((1, BLK, H), lambda i: (0, i, 0))
_spec_part1 = pl.BlockSpec((1, BLK, H), lambda i: (1, i, 0))
_spec_w = pl.BlockSpec((H, H), _const)
_spec_b = pl.BlockSpec((1, H), _const)

_sds_nh = jax.ShapeDtypeStruct((N, H), jnp.float32)

_tc0 = pl.pallas_call(
    _tc0_body,
    grid=(GRID,),
    in_specs=[pl.BlockSpec((BLK, 1), _nblk), _spec_w, _spec_w, _spec_w, _spec_b],
    out_specs=[_spec_nh, _spec_nh, _spec_nh],
    out_shape=[_sds_nh, _sds_nh, _sds_nh],
)

_tc_step = pl.pallas_call(
    _tc_step_body,
    grid=(GRID,),
    in_specs=[_spec_nh, _spec_nh, _spec_part0, _spec_part1, _spec_w, _spec_w, _spec_b],
    out_specs=[_spec_nh, _spec_nh, _spec_nh],
    out_shape=[_sds_nh, _sds_nh, _sds_nh],
)

_tc_final = pl.pallas_call(
    _tc_final_body,
    grid=(GRID,),
    in_specs=[_spec_nh, _spec_nh, _spec_part0, _spec_part1,
              _spec_w, _spec_b, _spec_w, _spec_b, _spec_w, _spec_b],
    out_specs=pl.BlockSpec((1, H), _const),
    out_shape=jax.ShapeDtypeStruct((1, H), jnp.float32),
    scratch_shapes=[pltpu.VMEM((1, H), jnp.float32)],
)


def _pad2(w, rows, cols):
    return jnp.zeros((rows, cols), jnp.float32).at[: w.shape[0], : w.shape[1]].set(w)


@jax.jit
def kernel(nfeats, edge_index, efeats, emb,
           W_self_0, W_agg_0, b_0,
           W_self_1, W_agg_1, b_1,
           W_self_2, W_agg_2, b_2,
           Wr0, br0, Wr1, br1, Wr2, br2):
    del efeats
    src = edge_index[0].reshape(NW, N_CHUNKS, CHUNK)
    dst = edge_index[1].reshape(NW, N_CHUNKS, CHUNK)
    nf = nfeats.reshape(N, 1)
    emb_p = _pad2(emb, H, H)
    zeros = jnp.zeros((WCHUNK, H), jnp.float32)

    wr0 = _pad2(Wr0, H, H)
    br0p = _pad2(br0.reshape(1, -1), 1, H)
    wr1 = _pad2(Wr1, H, H)
    br1p = _pad2(br1.reshape(1, -1), 1, H)
    wr2 = _pad2(Wr2, H, H)
    br2p = br2.reshape(1, H)

    h, u, v = _tc0(nf, emb_p, W_self_0, W_agg_0, b_0.reshape(1, H))
    p = _sc_segsum(src, dst, v, zeros)
    h, u, v = _tc_step(h, u, p, p, W_self_1, W_agg_1, b_1.reshape(1, H))
    p = _sc_segsum(src, dst, v, zeros)
    h, u, v = _tc_step(h, u, p, p, W_self_2, W_agg_2, b_2.reshape(1, H))
    p = _sc_segsum(src, dst, v, zeros)
    out = _tc_final(h, u, p, p, wr0, br0p, wr1, br1p, wr2, br2p)
    return out
